# Initial kernel scaffold; baseline (speedup 1.0000x reference)
import jax
import jax.numpy as jnp
from jax.experimental import pallas as pl


def kernel(points, features, mask, params):
    raise NotImplementedError("write your pallas kernel here")


# speedup vs baseline: 3.2112x; 3.2112x over previous
import jax
import jax.numpy as jnp
from jax.experimental import pallas as pl

_P = 128
_K = 7
_BLK = 1
_HI = jax.lax.Precision.HIGHEST


def _dot(a, b):
    return jax.lax.dot_general(a, b, (((1,), (0,)), ((), ())),
                               precision=_HI, preferred_element_type=jnp.float32)


def _dot_nt(a, b):
    return jax.lax.dot_general(a, b, (((1,), (1,)), ((), ())),
                               precision=_HI, preferred_element_type=jnp.float32)


def _topk_onehots(neg):
    iota = jax.lax.broadcasted_iota(jnp.int32, (_P, _P), 1)
    vals = neg
    ohs = []
    for i in range(_K + 1):
        mx = jnp.max(vals, axis=1, keepdims=True)
        cand = jnp.where(vals == mx, iota, _P)
        amin = jnp.min(cand, axis=1, keepdims=True)
        sel = iota == amin
        if i > 0:
            ohs.append(sel.astype(jnp.float32))
        if i < _K:
            vals = jnp.where(sel, -1e30, vals)
    return jnp.concatenate(ohs, axis=0)


def _edge_conv(neg, fts, Ws, bs, scW, scb, mT):
    ohs = _topk_onehots(neg)
    nbr = _dot(ohs, fts)
    ctr = jnp.concatenate([fts] * _K, axis=0)
    x = jnp.concatenate([ctr, nbr - ctr], axis=1)
    for W, b in zip(Ws, bs):
        x = jnp.maximum(_dot(x, W) + b, 0.0)
    agg = x[0:_P]
    for j in range(1, _K):
        agg = agg + x[j * _P:(j + 1) * _P]
    agg = agg * (1.0 / _K)
    sc = _dot(fts, scW) + scb
    return jnp.maximum(sc + agg, 0.0) * mT


def _pn_kernel(points_ref, features_ref, mask_ref,
               g16_ref, b16_ref,
               w10_ref, b10_ref, w11_ref, b11_ref, w12_ref, b12_ref,
               s1w_ref, s1b_ref,
               w20_ref, b20_ref, w21_ref, b21_ref, w22_ref, b22_ref,
               s2w_ref, s2b_ref,
               fusw_ref, fusb_ref, fcw_ref, fcb_ref,
               f1w_ref, f1b_ref, f2w_ref, f2b_ref,
               out_ref):
    for s in range(_BLK):
        m = mask_ref[s]
        mT = m.T
        shiftT = jnp.where(mT == 0.0, 1e9, 0.0)
        cnt = jnp.maximum(jnp.sum(m), 1.0)

        ftsT = features_ref[s].T * mT
        fts0 = (ftsT * g16_ref[...] + b16_ref[...]) * mT

        ptsC = points_ref[s] * m
        xr = ptsC[0:1] + shiftT.T
        yr = ptsC[1:2] + shiftT.T
        dx = xr.T - xr
        dy = yr.T - yr
        neg1 = -(dx * dx + dy * dy)
        fts1 = _edge_conv(neg1, fts0,
                          [w10_ref[...], w11_ref[...], w12_ref[...]],
                          [b10_ref[...], b11_ref[...], b12_ref[...]],
                          s1w_ref[...], s1b_ref[...], mT)

        pts2 = fts1 + shiftT
        gram = _dot_nt(pts2, pts2)
        xx = jnp.sum(pts2 * pts2, axis=1, keepdims=True)
        neg2 = 2.0 * gram - xx - xx.T
        fts2 = _edge_conv(neg2, fts1,
                          [w20_ref[...], w21_ref[...], w22_ref[...]],
                          [b20_ref[...], b21_ref[...], b22_ref[...]],
                          s2w_ref[...], s2b_ref[...], mT)

        cat = jnp.concatenate([fts1, fts2], axis=1)
        fused = jnp.maximum(_dot(cat, fusw_ref[...]) + fusb_ref[...], 0.0) * mT
        pooled = jnp.sum(fused, axis=0, keepdims=True) / cnt

        x = jnp.maximum(_dot(pooled, fcw_ref[...]) + fcb_ref[...], 0.0)
        x = _dot(x, f1w_ref[...]) + f1b_ref[...]
        x = jnp.where(x > 0.0, x, 0.01 * x)
        x = _dot(x, f2w_ref[...]) + f2b_ref[...]
        i = pl.program_id(0)
        out_ref[pl.ds(i * _BLK + s, 1), :] = x


def kernel(points, features, mask, params):
    p = params
    B = points.shape[0]
    inv = 1.0 / jnp.sqrt(jnp.float32(1.0 + 1e-5))

    def fold(W, g, b):
        return (W.T * (g * inv)[None, :]).astype(jnp.float32), b[None, :]

    weights = []
    weights.append((p['bn_fts_g'] * inv)[None, :])
    weights.append(p['bn_fts_b'][None, :])
    for i in range(3):
        W, b = fold(p['c1w%d' % i], p['c1g%d' % i], p['c1b%d' % i])
        weights += [W, b]
    W, b = fold(p['c1scw'], p['c1scg'], p['c1scb'])
    weights += [W, b]
    for i in range(3):
        W, b = fold(p['c2w%d' % i], p['c2g%d' % i], p['c2b%d' % i])
        weights += [W, b]
    W, b = fold(p['c2scw'], p['c2scg'], p['c2scb'])
    weights += [W, b]
    W, b = fold(p['fusw'], p['fusg'], p['fusb'])
    weights += [W, b]
    weights += [p['fcw'].T, p['fcb'][None, :]]
    weights += [p['fo1w'].T, p['fo1b'][None, :]]
    weights += [p['fo2w'].T, p['fo2b'][None, :]]

    in_specs = [
        pl.BlockSpec((_BLK, 2, _P), lambda i: (i, 0, 0)),
        pl.BlockSpec((_BLK, features.shape[1], _P), lambda i: (i, 0, 0)),
        pl.BlockSpec((_BLK, 1, _P), lambda i: (i, 0, 0)),
    ]
    for w in weights:
        in_specs.append(pl.BlockSpec(w.shape, lambda i: (0, 0)))

    out = pl.pallas_call(
        _pn_kernel,
        grid=(B // _BLK,),
        in_specs=in_specs,
        out_specs=pl.BlockSpec((B, 10), lambda i: (0, 0)),
        out_shape=jax.ShapeDtypeStruct((B, 10), jnp.float32),
    )(points, features, mask, *weights)
    return out


# speedup vs baseline: 17.3987x; 5.4182x over previous
import jax
import jax.numpy as jnp
from jax.experimental import pallas as pl

_P = 128
_K = 7
_BLK = 8
_HI = jax.lax.Precision.HIGHEST


def _dot(a, b):
    return jax.lax.dot_general(a, b, (((1,), (0,)), ((), ())),
                               preferred_element_type=jnp.float32)


def _dot_nt(a, b):
    return jax.lax.dot_general(a, b, (((1,), (1,)), ((), ())),
                               precision=_HI, preferred_element_type=jnp.float32)


def _dot_tn(a, b):
    return jax.lax.dot_general(a, b, (((0,), (0,)), ((), ())),
                               preferred_element_type=jnp.float32)


def _topk_onehots(neg):
    nrow = jax.lax.broadcasted_iota(jnp.int32, (_P, _P), 0)
    ki = jax.lax.bitcast_convert_type(neg - 1.0, jnp.int32)
    ki = (ki & jnp.int32(-128)) | nrow
    keys = jax.lax.bitcast_convert_type(ki, jnp.float32)
    ohs = []
    for i in range(_K + 1):
        mx = jnp.max(keys, axis=0, keepdims=True)
        sel = keys == mx
        if i > 0:
            ohs.append(sel.astype(jnp.float32))
        if i < _K:
            keys = jnp.where(sel, -jnp.inf, keys)
    return jnp.concatenate(ohs, axis=1)


def _edge_conv(neg, fts, Wc, Wn, b0, Ws, bs, scW, scb, mT):
    ohs = _topk_onehots(neg)
    base = _dot(fts, Wc) + b0
    nbrw = _dot_tn(ohs, _dot(fts, Wn))
    o = base.shape[1]
    x = jnp.maximum(nbrw.reshape(_K, _P, o) + base[None], 0.0)
    x = x.reshape(_K * _P, o)
    for W, b in zip(Ws, bs):
        x = jnp.maximum(_dot(x, W) + b, 0.0)
    agg = jnp.sum(x.reshape(_K, _P, o), axis=0) * (1.0 / _K)
    sc = _dot(fts, scW) + scb
    return jnp.maximum(sc + agg, 0.0) * mT


def _pn_kernel(points_ref, features_ref, mask_ref,
               g16_ref, b16_ref,
               wc1_ref, wn1_ref, b10_ref, w11_ref, b11_ref, w12_ref, b12_ref,
               s1w_ref, s1b_ref,
               wc2_ref, wn2_ref, b20_ref, w21_ref, b21_ref, w22_ref, b22_ref,
               s2w_ref, s2b_ref,
               fusw_ref, fusb_ref, fcw_ref, fcb_ref,
               f1w_ref, f1b_ref, f2w_ref, f2b_ref,
               out_ref):
    for s in range(_BLK):
        m = mask_ref[s]
        mT = m.T
        shiftT = jnp.where(mT == 0.0, 1e9, 0.0)
        cnt = jnp.maximum(jnp.sum(m), 1.0)

        ftsT = features_ref[s] * mT
        fts0 = (ftsT * g16_ref[...] + b16_ref[...]) * mT

        ptsC = points_ref[s] * m
        xr = ptsC[0:1] + shiftT.T
        yr = ptsC[1:2] + shiftT.T
        dx = xr.T - xr
        dy = yr.T - yr
        neg1 = -(dx * dx + dy * dy)
        fts1 = _edge_conv(neg1, fts0,
                          wc1_ref[...], wn1_ref[...], b10_ref[...],
                          [w11_ref[...], w12_ref[...]],
                          [b11_ref[...], b12_ref[...]],
                          s1w_ref[...], s1b_ref[...], mT)

        pts2 = fts1 + shiftT
        gram = _dot_nt(pts2, pts2)
        xx = jnp.sum(pts2 * pts2, axis=1, keepdims=True)
        neg2 = 2.0 * gram - xx - xx.T
        fts2 = _edge_conv(neg2, fts1,
                          wc2_ref[...], wn2_ref[...], b20_ref[...],
                          [w21_ref[...], w22_ref[...]],
                          [b21_ref[...], b22_ref[...]],
                          s2w_ref[...], s2b_ref[...], mT)

        cat = jnp.concatenate([fts1, fts2], axis=1)
        fused = jnp.maximum(_dot(cat, fusw_ref[...]) + fusb_ref[...], 0.0) * mT
        pooled = jnp.sum(fused, axis=0, keepdims=True) / cnt

        x = jnp.maximum(_dot(pooled, fcw_ref[...]) + fcb_ref[...], 0.0)
        x = _dot(x, f1w_ref[...]) + f1b_ref[...]
        x = jnp.where(x > 0.0, x, 0.01 * x)
        x = _dot(x, f2w_ref[...]) + f2b_ref[...]
        i = pl.program_id(0)
        out_ref[pl.ds(i * _BLK + s, 1), :] = x


def kernel(points, features, mask, params):
    p = params
    B = points.shape[0]
    inv = 1.0 / jnp.sqrt(jnp.float32(1.0 + 1e-5))

    def fold(W, g, b):
        return (W.T * (g * inv)[None, :]).astype(jnp.float32), b[None, :]

    weights = []
    weights.append((p['bn_fts_g'] * inv)[None, :])
    weights.append(p['bn_fts_b'][None, :])
    for blk in ('c1', 'c2'):
        W0, b0 = fold(p[blk + 'w0'], p[blk + 'g0'], p[blk + 'b0'])
        c = W0.shape[0] // 2
        weights += [W0[:c] - W0[c:], W0[c:], b0]
        for i in (1, 2):
            W, b = fold(p['%sw%d' % (blk, i)], p['%sg%d' % (blk, i)],
                        p['%sb%d' % (blk, i)])
            weights += [W, b]
        W, b = fold(p[blk + 'scw'], p[blk + 'scg'], p[blk + 'scb'])
        weights += [W, b]
    W, b = fold(p['fusw'], p['fusg'], p['fusb'])
    weights += [W, b]
    weights += [p['fcw'].T, p['fcb'][None, :]]
    weights += [p['fo1w'].T, p['fo1b'][None, :]]
    weights += [p['fo2w'].T, p['fo2b'][None, :]]

    in_specs = [
        pl.BlockSpec((_BLK, 2, _P), lambda i: (i, 0, 0)),
        pl.BlockSpec((_BLK, _P, features.shape[1]), lambda i: (i, 0, 0)),
        pl.BlockSpec((_BLK, 1, _P), lambda i: (i, 0, 0)),
    ]
    for w in weights:
        in_specs.append(pl.BlockSpec(w.shape, lambda i: (0, 0)))

    out = pl.pallas_call(
        _pn_kernel,
        grid=(B // _BLK,),
        in_specs=in_specs,
        out_specs=pl.BlockSpec((B, 10), lambda i: (0, 0)),
        out_shape=jax.ShapeDtypeStruct((B, 10), jnp.float32),
    )(points, jnp.transpose(features, (0, 2, 1)), mask, *weights)
    return out


# speedup vs baseline: 17.7681x; 1.0212x over previous
import jax
import jax.numpy as jnp
from jax.experimental import pallas as pl
from jax.experimental.pallas import tpu as pltpu

_P = 128
_K = 7
_BLK = 16
_HI = jax.lax.Precision.HIGHEST


def _dot(a, b):
    return jax.lax.dot_general(a, b, (((1,), (0,)), ((), ())),
                               preferred_element_type=jnp.float32)


def _dot_nt(a, b):
    return jax.lax.dot_general(a, b, (((1,), (1,)), ((), ())),
                               precision=_HI, preferred_element_type=jnp.float32)


def _dot_tn(a, b):
    return jax.lax.dot_general(a, b, (((0,), (0,)), ((), ())),
                               preferred_element_type=jnp.float32)


def _topk_onehots(neg):
    nrow = jax.lax.broadcasted_iota(jnp.int32, (_P, _P), 0)
    ki = jax.lax.bitcast_convert_type(neg - 1.0, jnp.int32)
    ki = (ki & jnp.int32(-128)) | nrow
    keys = jax.lax.bitcast_convert_type(ki, jnp.float32)
    ohs = []
    for i in range(_K + 1):
        mx = jnp.max(keys, axis=0, keepdims=True)
        sel = keys == mx
        if i > 0:
            ohs.append(sel.astype(jnp.float32))
        if i < _K:
            keys = jnp.where(sel, -jnp.inf, keys)
    return jnp.concatenate(ohs, axis=1)


def _edge_conv(neg, fts, Wc, Wn, b0, Ws, bs, scW, scb, mT):
    ohs = _topk_onehots(neg)
    base = _dot(fts, Wc) + b0
    nbrw = _dot_tn(ohs, _dot(fts, Wn))
    o = base.shape[1]
    x = jnp.maximum(nbrw.reshape(_K, _P, o) + base[None], 0.0)
    x = x.reshape(_K * _P, o)
    for W, b in zip(Ws, bs):
        x = jnp.maximum(_dot(x, W) + b, 0.0)
    agg = jnp.sum(x.reshape(_K, _P, o), axis=0) * (1.0 / _K)
    sc = _dot(fts, scW) + scb
    return jnp.maximum(sc + agg, 0.0) * mT


def _pn_kernel(points_ref, features_ref, mask_ref,
               g16_ref, b16_ref,
               wc1_ref, wn1_ref, b10_ref, w11_ref, b11_ref, w12_ref, b12_ref,
               s1w_ref, s1b_ref,
               wc2_ref, wn2_ref, b20_ref, w21_ref, b21_ref, w22_ref, b22_ref,
               s2w_ref, s2b_ref,
               fusw_ref, fusb_ref, fcw_ref, fcb_ref,
               f1w_ref, f1b_ref, f2w_ref, f2b_ref,
               out_ref):
    for s in range(_BLK):
        m = mask_ref[s]
        mT = m.T
        shiftT = jnp.where(mT == 0.0, 1e9, 0.0)
        cnt = jnp.maximum(jnp.sum(m), 1.0)

        ftsT = features_ref[s] * mT
        fts0 = (ftsT * g16_ref[...] + b16_ref[...]) * mT

        ptsC = points_ref[s] * m
        xr = ptsC[0:1] + shiftT.T
        yr = ptsC[1:2] + shiftT.T
        dx = xr.T - xr
        dy = yr.T - yr
        neg1 = -(dx * dx + dy * dy)
        fts1 = _edge_conv(neg1, fts0,
                          wc1_ref[...], wn1_ref[...], b10_ref[...],
                          [w11_ref[...], w12_ref[...]],
                          [b11_ref[...], b12_ref[...]],
                          s1w_ref[...], s1b_ref[...], mT)

        pts2 = fts1 + shiftT
        gram = _dot_nt(pts2, pts2)
        xx = jnp.sum(pts2 * pts2, axis=1, keepdims=True)
        neg2 = 2.0 * gram - xx - xx.T
        fts2 = _edge_conv(neg2, fts1,
                          wc2_ref[...], wn2_ref[...], b20_ref[...],
                          [w21_ref[...], w22_ref[...]],
                          [b21_ref[...], b22_ref[...]],
                          s2w_ref[...], s2b_ref[...], mT)

        cat = jnp.concatenate([fts1, fts2], axis=1)
        fused = jnp.maximum(_dot(cat, fusw_ref[...]) + fusb_ref[...], 0.0) * mT
        pooled = jnp.sum(fused, axis=0, keepdims=True) / cnt

        x = jnp.maximum(_dot(pooled, fcw_ref[...]) + fcb_ref[...], 0.0)
        x = _dot(x, f1w_ref[...]) + f1b_ref[...]
        x = jnp.where(x > 0.0, x, 0.01 * x)
        x = _dot(x, f2w_ref[...]) + f2b_ref[...]
        out_ref[pl.ds(s, 1), :] = x


def kernel(points, features, mask, params):
    p = params
    B = points.shape[0]
    inv = 1.0 / jnp.sqrt(jnp.float32(1.0 + 1e-5))

    def fold(W, g, b):
        return (W.T * (g * inv)[None, :]).astype(jnp.float32), b[None, :]

    weights = []
    weights.append((p['bn_fts_g'] * inv)[None, :])
    weights.append(p['bn_fts_b'][None, :])
    for blk in ('c1', 'c2'):
        W0, b0 = fold(p[blk + 'w0'], p[blk + 'g0'], p[blk + 'b0'])
        c = W0.shape[0] // 2
        weights += [W0[:c] - W0[c:], W0[c:], b0]
        for i in (1, 2):
            W, b = fold(p['%sw%d' % (blk, i)], p['%sg%d' % (blk, i)],
                        p['%sb%d' % (blk, i)])
            weights += [W, b]
        W, b = fold(p[blk + 'scw'], p[blk + 'scg'], p[blk + 'scb'])
        weights += [W, b]
    W, b = fold(p['fusw'], p['fusg'], p['fusb'])
    weights += [W, b]
    weights += [p['fcw'].T, p['fcb'][None, :]]
    weights += [p['fo1w'].T, p['fo1b'][None, :]]
    weights += [p['fo2w'].T, p['fo2b'][None, :]]

    in_specs = [
        pl.BlockSpec((_BLK, 2, _P), lambda i: (i, 0, 0)),
        pl.BlockSpec((_BLK, _P, features.shape[1]), lambda i: (i, 0, 0)),
        pl.BlockSpec((_BLK, 1, _P), lambda i: (i, 0, 0)),
    ]
    for w in weights:
        in_specs.append(pl.BlockSpec(w.shape, lambda i: (0, 0)))

    out = pl.pallas_call(
        _pn_kernel,
        grid=(B // _BLK,),
        in_specs=in_specs,
        out_specs=pl.BlockSpec((_BLK, 10), lambda i: (i, 0)),
        out_shape=jax.ShapeDtypeStruct((B, 10), jnp.float32),
        compiler_params=pltpu.CompilerParams(
            dimension_semantics=("parallel",)),
    )(points, jnp.transpose(features, (0, 2, 1)), mask, *weights)
    return out
